# baseline (speedup 1.0000x reference)
import jax
import jax.numpy as jnp
from jax.experimental import pallas as pl
from jax.experimental.pallas import tpu as pltpu

_B = 4096
_D = 128
_S = 1024
_NDMA = _B // _S


def _tc_body(times_ref, mask_ref, out_ref, buf, sem):
    t = times_ref[0]
    row = pltpu.make_async_copy(
        mask_ref.at[pl.ds(t, 1)], buf.at[pl.ds(0, 1)], sem
    )
    row.start()
    row.wait()
    buf[...] = jnp.broadcast_to(buf[pl.ds(0, 1), :], (_S, _D))
    copies = [
        pltpu.make_async_copy(buf, out_ref.at[pl.ds(k * _S, _S)], sem)
        for k in range(_NDMA)
    ]
    for c in copies:
        c.start()
    for c in copies:
        c.wait()


def _make_call(interpret: bool = False):
    return pl.pallas_call(
        _tc_body,
        grid=(1,),
        in_specs=[
            pl.BlockSpec(memory_space=pltpu.SMEM),
            pl.BlockSpec(memory_space=pl.ANY),
        ],
        out_specs=pl.BlockSpec(memory_space=pl.ANY),
        scratch_shapes=[
            pltpu.VMEM((_S, _D), jnp.float32),
            pltpu.SemaphoreType.DMA,
        ],
        out_shape=jax.ShapeDtypeStruct((_B, _D), jnp.float32),
        interpret=interpret,
    )


def kernel(inps, times, mask):
    del inps
    t = jnp.atleast_1d(jnp.asarray(times, dtype=jnp.int32))
    return _make_call()(t, mask)


# speedup vs baseline: 1.0003x; 1.0003x over previous
import jax
import jax.numpy as jnp
from jax.experimental import pallas as pl
from jax.experimental.pallas import tpu as pltpu

_B = 4096
_D = 128
_S = 512
_NDMA = _B // _S


def _tc_body(times_ref, mask_ref, out_ref, buf, sem):
    t = times_ref[0]
    row = pltpu.make_async_copy(
        mask_ref.at[pl.ds(t, 1)], buf.at[pl.ds(0, 1)], sem
    )
    row.start()
    row.wait()
    buf[...] = jnp.broadcast_to(buf[pl.ds(0, 1), :], (_S, _D))
    copies = [
        pltpu.make_async_copy(buf, out_ref.at[pl.ds(k * _S, _S)], sem)
        for k in range(_NDMA)
    ]
    for c in copies:
        c.start()
    for c in copies:
        c.wait()


def _make_call(interpret: bool = False):
    return pl.pallas_call(
        _tc_body,
        in_specs=[
            pl.BlockSpec(memory_space=pltpu.SMEM),
            pl.BlockSpec(memory_space=pl.ANY),
        ],
        out_specs=pl.BlockSpec(memory_space=pl.ANY),
        scratch_shapes=[
            pltpu.VMEM((_S, _D), jnp.float32),
            pltpu.SemaphoreType.DMA,
        ],
        out_shape=jax.ShapeDtypeStruct((_B, _D), jnp.float32),
        interpret=interpret,
    )


def kernel(inps, times, mask):
    del inps
    t = jnp.atleast_1d(jnp.asarray(times, dtype=jnp.int32))
    return _make_call()(t, mask)


# speedup vs baseline: 1.0122x; 1.0119x over previous
import jax
import jax.numpy as jnp
from jax.experimental import pallas as pl
from jax.experimental.pallas import tpu as pltpu

_B = 4096
_D = 128
_S = 512
_NDMA = _B // _S


def _tc_body(times_ref, mask_ref, out_ref, buf, sem, sem2):
    t = times_ref[0]
    row = pltpu.make_async_copy(
        mask_ref.at[pl.ds(t, 1)], buf.at[pl.ds(0, 1)], sem
    )
    row.start()
    row.wait()
    buf[...] = jnp.broadcast_to(buf[pl.ds(0, 1), :], (_S, _D))
    copies = [
        pltpu.make_async_copy(
            buf, out_ref.at[pl.ds(k * _S, _S)], sem if k % 2 == 0 else sem2
        )
        for k in range(_NDMA)
    ]
    for c in copies:
        c.start()
    for c in copies:
        c.wait()


def _make_call(interpret: bool = False):
    return pl.pallas_call(
        _tc_body,
        in_specs=[
            pl.BlockSpec(memory_space=pltpu.SMEM),
            pl.BlockSpec(memory_space=pl.ANY),
        ],
        out_specs=pl.BlockSpec(memory_space=pl.ANY),
        scratch_shapes=[
            pltpu.VMEM((_S, _D), jnp.float32),
            pltpu.SemaphoreType.DMA,
            pltpu.SemaphoreType.DMA,
        ],
        out_shape=jax.ShapeDtypeStruct((_B, _D), jnp.float32),
        interpret=interpret,
    )


def kernel(inps, times, mask):
    del inps
    t = jnp.atleast_1d(jnp.asarray(times, dtype=jnp.int32))
    return _make_call()(t, mask)
